# baseline (speedup 1.0000x reference)
import jax
import jax.numpy as jnp
from jax import lax
from jax.experimental import pallas as pl
from jax.experimental.pallas import tpu as pltpu
from jax.experimental.pallas import tpu_sc as plsc

M = 100000
D = 128
B = 16384
NC = 2
NS = 16
NW = NC * NS

RPW = 3136
LAST_ROWS = M - (NW - 1) * RPW
PT = RPW
BPW = B // NW
GCH = 128
SCH = 128
CCH = 128
CCT = 32
NSV = B // 16
NCH_MAX = (RPW + SCH - 1) // SCH
LIST_CAP = RPW + 2 * SCH


def _body(mem, values, ts, ids, gathered, new_lu,
          ids_v, ts_v, last_b, lu_v, winb_flat, dstr_flat, dstr2d,
          gbuf, sbuf, cb0, cb1,
          sem_t, sem_g, sg0, sg1, ss0, ss1):
    wid = lax.axis_index("s") * NC + lax.axis_index("c")
    base_r = wid * RPW
    is_last = wid == NW - 1
    nrows = jnp.where(is_last, LAST_ROWS, RPW).astype(jnp.int32)
    nvec = nrows // 16
    iota = lax.broadcasted_iota(jnp.int32, (16,), 0)

    ncc = nrows // CCH
    ntail = (nrows - ncc * CCH) // CCT

    def cgather(c, buf, sem):
        pltpu.async_copy(mem.at[pl.ds(base_r + c * CCH, CCH)], buf, sem)

    def cscatter(c, buf, sem):
        pltpu.async_copy(buf, new_mem.at[pl.ds(base_r + c * CCH, CCH)], sem)

    def cwait_g(c, buf, sem):
        pltpu.make_async_copy(mem.at[pl.ds(base_r + c * CCH, CCH)], buf, sem).wait()

    def cwait_s(c, buf, sem):
        pltpu.make_async_copy(buf, new_mem.at[pl.ds(base_r + c * CCH, CCH)],
                              sem).wait()

    def init_body(i, _):
        last_b[pl.ds(i * 16, 16)] = jnp.full((16,), -1, jnp.int32)
        return 0
    lax.fori_loop(0, PT // 16, init_body, 0)

    def scan_body(i, _):
        ids16 = ids_v[pl.ds(i * 16, 16)]
        mine = (ids16 >= base_r) & (ids16 < base_r + nrows)
        plsc.store_scatter(last_b, [ids16 - base_r], i * 16 + iota, mask=mine)
        return 0

    pltpu.sync_copy(ids, ids_v)
    cp_ts = pltpu.async_copy(ts, ts_v, sem_t)
    lax.fori_loop(0, NSV, scan_body, 0)
    gbase = wid * BPW

    def gfire(k, buf, sem):
        pltpu.async_copy(mem.at[ids_v.at[pl.ds(gbase + k * GCH, GCH)]], buf, sem)

    def gwait(k, buf, sem):
        pltpu.make_async_copy(mem.at[ids_v.at[pl.ds(gbase + k * GCH, GCH)]],
                              buf, sem).wait()

    gfire(0, gbuf, sg0)
    gfire(1, sbuf, sg1)

    cp_ts.wait()

    def tbl_body(i, cnt):
        lb = last_b[pl.ds(i * 16, 16)]
        m = lb >= 0
        t = plsc.load_gather(ts_v, [lb], mask=m)
        lu_v[pl.ds(i * 16, 16)] = jnp.where(m, t, jnp.float32(0.0))
        plsc.store_compressed(winb_flat.at[pl.ds(cnt, 16)], lb, mask=m)
        grow = base_r + i * 16 + iota
        plsc.store_compressed(dstr_flat.at[pl.ds(cnt, 16)], grow, mask=m)
        return cnt + jnp.sum(m.astype(jnp.int32))
    with jax.named_scope("p4_sweep"):
        cnt = lax.fori_loop(0, nvec, tbl_body, jnp.int32(0))

    gwait(0, gbuf, sg0)
    pltpu.sync_copy(gbuf, gathered.at[pl.ds(gbase, GCH)])
    gfire(2, gbuf, sg0)
    gwait(1, sbuf, sg1)
    pltpu.sync_copy(sbuf, gathered.at[pl.ds(gbase + GCH, GCH)])
    gfire(3, sbuf, sg1)

    @pl.when(cnt > 0)
    def _():
        lastix = jnp.full((16,), cnt - 1, jnp.int32)
        wpad = plsc.load_gather(winb_flat, [lastix])
        dpad = plsc.load_gather(dstr_flat, [lastix])
        for k in range(SCH // 16):
            winb_flat[pl.ds(cnt + k * 16, 16)] = wpad
            dstr_flat[pl.ds(cnt + k * 16, 16)] = dpad

    nch = (cnt + SCH - 1) // SCH

    def tr_body(j, _):
        v = dstr_flat[pl.ds(j * 16, 16)]
        dstr2d[j // 8, pl.ds((j % 8) * 16, 16)] = v
        return 0
    lax.fori_loop(0, nch * (SCH // 16), tr_body, 0)

    @pl.when(jnp.logical_not(is_last))
    def _():
        pltpu.sync_copy(lu_v.at[pl.ds(0, RPW)], new_lu.at[pl.ds(base_r, RPW)])

    @pl.when(is_last)
    def _():
        pltpu.sync_copy(lu_v.at[pl.ds(0, LAST_ROWS)],
                        new_lu.at[pl.ds(base_r, LAST_ROWS)])

    with jax.named_scope("p5_gout"):
        gwait(2, gbuf, sg0)
        pltpu.sync_copy(gbuf, gathered.at[pl.ds(gbase + 2 * GCH, GCH)])
        gwait(3, sbuf, sg1)
        pltpu.sync_copy(sbuf, gathered.at[pl.ds(gbase + 3 * GCH, GCH)])


def _tc_copy_body(x_ref, o_ref):
    o_ref[...] = x_ref[...]


def _tc_copy(mem):
    TCB = 1000
    return pl.pallas_call(
        _tc_copy_body,
        out_shape=jax.ShapeDtypeStruct((M, D), jnp.float32),
        grid=(M // TCB,),
        in_specs=[pl.BlockSpec((TCB, D), lambda i: (i, 0))],
        out_specs=pl.BlockSpec((TCB, D), lambda i: (i, 0)),
    )(mem)


def kernel(mem, values, timestamps, node_ids):
    mesh = plsc.VectorSubcoreMesh(core_axis_name="c", subcore_axis_name="s")
    out = pl.kernel(
        _body,
        out_type=(
            jax.ShapeDtypeStruct((B, D), jnp.float32),
            jax.ShapeDtypeStruct((M,), jnp.float32),
        ),
        mesh=mesh,
        compiler_params=pltpu.CompilerParams(needs_layout_passes=False),
        scratch_types=[
            pltpu.VMEM((B,), jnp.int32),
            pltpu.VMEM((B,), jnp.float32),
            pltpu.VMEM((PT,), jnp.int32),
            pltpu.VMEM((PT,), jnp.float32),
            pltpu.VMEM((LIST_CAP,), jnp.int32),
            pltpu.VMEM((LIST_CAP,), jnp.int32),
            pltpu.VMEM((NCH_MAX, SCH), jnp.int32),
            pltpu.VMEM((GCH, D), jnp.float32),
            pltpu.VMEM((SCH, D), jnp.float32),
            pltpu.VMEM((CCH, D), jnp.float32),
            pltpu.VMEM((CCH, D), jnp.float32),
            pltpu.SemaphoreType.DMA,
            pltpu.SemaphoreType.DMA,
            pltpu.SemaphoreType.DMA,
            pltpu.SemaphoreType.DMA,
            pltpu.SemaphoreType.DMA,
            pltpu.SemaphoreType.DMA,
        ],
    )(mem, values, timestamps, node_ids)
    gathered, new_lu = out
    new_mem = _tc_copy(mem)
    return gathered, new_mem, new_lu


# speedup vs baseline: 1.0843x; 1.0843x over previous
import jax
import jax.numpy as jnp
from jax import lax
from jax.experimental import pallas as pl
from jax.experimental.pallas import tpu as pltpu
from jax.experimental.pallas import tpu_sc as plsc

M = 100000
D = 128
B = 16384
NC = 2
NS = 16
NW = NC * NS

RPW = 3136
LAST_ROWS = M - (NW - 1) * RPW
PT = RPW
BPW = B // NW
GCH = 128
SCH = 128
CCH = 128
CCT = 32
NSV = B // 16
NCH_MAX = (RPW + SCH - 1) // SCH
LIST_CAP = RPW + 2 * SCH


def _body(mem, values, ts, ids, gathered, new_mem, new_lu,
          ids_v, ts_v, last_b, lu_v, winb_flat, dstr_flat, dstr2d,
          gbuf, sbuf, cb0, cb1,
          sem_t, sem_g, sem_g2, sg0, sg1, ss0, ss1):
    wid = lax.axis_index("s") * NC + lax.axis_index("c")
    base_r = wid * RPW
    is_last = wid == NW - 1
    nrows = jnp.where(is_last, LAST_ROWS, RPW).astype(jnp.int32)
    nvec = nrows // 16
    iota = lax.broadcasted_iota(jnp.int32, (16,), 0)

    ncc = nrows // CCH
    ntail = (nrows - ncc * CCH) // CCT
    hc = ncc // 2

    def cgather(c, buf, sem):
        pltpu.async_copy(mem.at[pl.ds(base_r + c * CCH, CCH)], buf, sem)

    def cscatter(c, buf, sem):
        pltpu.async_copy(buf, new_mem.at[pl.ds(base_r + c * CCH, CCH)], sem)

    def cwait_g(c, buf, sem):
        pltpu.make_async_copy(mem.at[pl.ds(base_r + c * CCH, CCH)], buf, sem).wait()

    def cwait_s(c, buf, sem):
        pltpu.make_async_copy(buf, new_mem.at[pl.ds(base_r + c * CCH, CCH)],
                              sem).wait()

    gbase = wid * BPW

    def gfire(k, buf, sem):
        pltpu.async_copy(mem.at[ids_v.at[pl.ds(gbase + k * GCH, GCH)]], buf, sem)

    def gwait(k, buf, sem):
        pltpu.make_async_copy(mem.at[ids_v.at[pl.ds(gbase + k * GCH, GCH)]],
                              buf, sem).wait()

    def gout(k, buf):
        pltpu.sync_copy(buf, gathered.at[pl.ds(gbase + k * GCH, GCH)])

    cgather(0, cb0, sg0)
    pltpu.async_copy(ts, ts_v, sem_t)
    pltpu.sync_copy(ids, ids_v)
    gfire(0, gbuf, sem_g)
    gfire(1, sbuf, sem_g2)

    def init_body(i, _):
        last_b[pl.ds(i * 16, 16)] = jnp.full((16,), -1, jnp.int32)
        return 0
    lax.fori_loop(0, PT // 16, init_body, 0)

    def scan_body(i, _):
        ids16 = ids_v[pl.ds(i * 16, 16)]
        mine = (ids16 >= base_r) & (ids16 < base_r + nrows)
        plsc.store_scatter(last_b, [ids16 - base_r], i * 16 + iota, mask=mine)
        return 0

    def tbl_body(i, cnt):
        lb = last_b[pl.ds(i * 16, 16)]
        m = lb >= 0
        t = plsc.load_gather(ts_v, [lb], mask=m)
        lu_v[pl.ds(i * 16, 16)] = jnp.where(m, t, jnp.float32(0.0))
        plsc.store_compressed(winb_flat.at[pl.ds(cnt, 16)], lb, mask=m)
        grow = base_r + i * 16 + iota
        plsc.store_compressed(dstr_flat.at[pl.ds(cnt, 16)], grow, mask=m)
        return cnt + jnp.sum(m.astype(jnp.int32))

    def copy_body(c, cnt):
        @pl.when(c % 2 == 0)
        def _():
            @pl.when(c + 1 < ncc)
            def _():
                @pl.when(c >= 1)
                def _():
                    cwait_s(c - 1, cb1, ss1)
                cgather(c + 1, cb1, sg1)

        @pl.when(c % 2 == 1)
        def _():
            @pl.when(c + 1 < ncc)
            def _():
                cwait_s(c - 1, cb0, ss0)
                cgather(c + 1, cb0, sg0)

        sl = jnp.minimum(c, hc) * NSV // hc
        sh = jnp.minimum(c + 1, hc) * NSV // hc
        lax.fori_loop(sl, sh, scan_body, 0)

        @pl.when(c == hc)
        def _():
            pltpu.make_async_copy(ts, ts_v, sem_t).wait()
            gwait(0, gbuf, sem_g)
            gout(0, gbuf)
            gfire(2, gbuf, sem_g)

        @pl.when(c == hc + 3)
        def _():
            gwait(1, sbuf, sem_g2)
            gout(1, sbuf)
            gfire(3, sbuf, sem_g2)

        swl = (jnp.maximum(c, hc) - hc) * nvec // (ncc - hc)
        swh = (jnp.maximum(c + 1, hc) - hc) * nvec // (ncc - hc)
        cnt = lax.fori_loop(swl, swh, tbl_body, cnt)

        @pl.when(c % 2 == 0)
        def _():
            cwait_g(c, cb0, sg0)
            cscatter(c, cb0, ss0)

        @pl.when(c % 2 == 1)
        def _():
            cwait_g(c, cb1, sg1)
            cscatter(c, cb1, ss1)
        return cnt
    cnt = lax.fori_loop(0, ncc, copy_body, jnp.int32(0))

    @pl.when(ncc % 2 == 0)
    def _():
        cwait_s(ncc - 2, cb0, ss0)
        cwait_s(ncc - 1, cb1, ss1)

    @pl.when(ncc % 2 == 1)
    def _():
        cwait_s(ncc - 2, cb1, ss1)
        cwait_s(ncc - 1, cb0, ss0)

    tbase = base_r + ncc * CCH

    def tail_body(t, _):
        pltpu.async_copy(mem.at[pl.ds(tbase + t * CCT, CCT)],
                         cb0.at[pl.ds(0, CCT)], sg0).wait()
        pltpu.async_copy(cb0.at[pl.ds(0, CCT)],
                         new_mem.at[pl.ds(tbase + t * CCT, CCT)], ss0).wait()
        return 0
    lax.fori_loop(0, ntail, tail_body, 0)

    gwait(2, gbuf, sem_g)
    gout(2, gbuf)
    gwait(3, sbuf, sem_g2)
    gout(3, sbuf)

    @pl.when(cnt > 0)
    def _():
        lastix = jnp.full((16,), cnt - 1, jnp.int32)
        wpad = plsc.load_gather(winb_flat, [lastix])
        dpad = plsc.load_gather(dstr_flat, [lastix])
        for k in range(SCH // 16):
            winb_flat[pl.ds(cnt + k * 16, 16)] = wpad
            dstr_flat[pl.ds(cnt + k * 16, 16)] = dpad

    nch = (cnt + SCH - 1) // SCH

    def tr_body(j, _):
        v = dstr_flat[pl.ds(j * 16, 16)]
        dstr2d[j // 8, pl.ds((j % 8) * 16, 16)] = v
        return 0
    lax.fori_loop(0, nch * (SCH // 16), tr_body, 0)

    @pl.when(jnp.logical_not(is_last))
    def _():
        pltpu.sync_copy(lu_v.at[pl.ds(0, RPW)], new_lu.at[pl.ds(base_r, RPW)])

    @pl.when(is_last)
    def _():
        pltpu.sync_copy(lu_v.at[pl.ds(0, LAST_ROWS)],
                        new_lu.at[pl.ds(base_r, LAST_ROWS)])

    def vg(c, buf, sem):
        pltpu.async_copy(values.at[winb_flat.at[pl.ds(c * SCH, SCH)]], buf, sem)

    def vgw(c, buf, sem):
        pltpu.make_async_copy(values.at[winb_flat.at[pl.ds(c * SCH, SCH)]],
                              buf, sem).wait()

    def rs(c, buf, sem):
        pltpu.async_copy(buf, new_mem.at[dstr2d.at[c]], sem)

    def rsw(c, buf, sem):
        pltpu.make_async_copy(buf, new_mem.at[dstr2d.at[c]], sem).wait()

    @pl.when(nch > 0)
    def _():
        vg(0, gbuf, sg0)

        def sc_body(c, _):
            @pl.when(c % 2 == 0)
            def _():
                @pl.when(c + 1 < nch)
                def _():
                    @pl.when(c >= 1)
                    def _():
                        rsw(c - 1, sbuf, ss1)
                    vg(c + 1, sbuf, sg1)
                vgw(c, gbuf, sg0)
                rs(c, gbuf, ss0)

            @pl.when(c % 2 == 1)
            def _():
                @pl.when(c + 1 < nch)
                def _():
                    rsw(c - 1, gbuf, ss0)
                    vg(c + 1, gbuf, sg0)
                vgw(c, sbuf, sg1)
                rs(c, sbuf, ss1)
            return 0
        lax.fori_loop(0, nch, sc_body, 0)

        @pl.when(nch == 1)
        def _():
            rsw(0, gbuf, ss0)

        @pl.when((nch > 1) & (nch % 2 == 0))
        def _():
            rsw(nch - 2, gbuf, ss0)
            rsw(nch - 1, sbuf, ss1)

        @pl.when((nch > 1) & (nch % 2 == 1))
        def _():
            rsw(nch - 2, sbuf, ss1)
            rsw(nch - 1, gbuf, ss0)


def kernel(mem, values, timestamps, node_ids):
    mesh = plsc.VectorSubcoreMesh(core_axis_name="c", subcore_axis_name="s")
    out = pl.kernel(
        _body,
        out_type=(
            jax.ShapeDtypeStruct((B, D), jnp.float32),
            jax.ShapeDtypeStruct((M, D), jnp.float32),
            jax.ShapeDtypeStruct((M,), jnp.float32),
        ),
        mesh=mesh,
        compiler_params=pltpu.CompilerParams(needs_layout_passes=False),
        scratch_types=[
            pltpu.VMEM((B,), jnp.int32),
            pltpu.VMEM((B,), jnp.float32),
            pltpu.VMEM((PT,), jnp.int32),
            pltpu.VMEM((PT,), jnp.float32),
            pltpu.VMEM((LIST_CAP,), jnp.int32),
            pltpu.VMEM((LIST_CAP,), jnp.int32),
            pltpu.VMEM((NCH_MAX, SCH), jnp.int32),
            pltpu.VMEM((GCH, D), jnp.float32),
            pltpu.VMEM((SCH, D), jnp.float32),
            pltpu.VMEM((CCH, D), jnp.float32),
            pltpu.VMEM((CCH, D), jnp.float32),
            pltpu.SemaphoreType.DMA,
            pltpu.SemaphoreType.DMA,
            pltpu.SemaphoreType.DMA,
            pltpu.SemaphoreType.DMA,
            pltpu.SemaphoreType.DMA,
            pltpu.SemaphoreType.DMA,
            pltpu.SemaphoreType.DMA,
        ],
    )(mem, values, timestamps, node_ids)
    return out


# speedup vs baseline: 1.1021x; 1.0164x over previous
import jax
import jax.numpy as jnp
from jax import lax
from jax.experimental import pallas as pl
from jax.experimental.pallas import tpu as pltpu
from jax.experimental.pallas import tpu_sc as plsc

M = 100000
D = 128
B = 16384
NC = 2
NS = 16
NW = NC * NS

RPW = 3136
LAST_ROWS = M - (NW - 1) * RPW
PT = RPW
BPW = B // NW
GCH = 128
SCH = 128
CCH = 128
CCT = 32
NSV = B // 16
NCH_MAX = (RPW + SCH - 1) // SCH
LIST_CAP = RPW + 2 * SCH


def _body(mem, values, ts, ids, gathered, new_mem, new_lu,
          ids_v, ts_v, last_b, lu_v, winb_flat, dstr_flat, dstr2d,
          gbuf, sbuf, cb0, cb1,
          sem_t, sem_g, sem_g2, sg0, sg1, ss0, ss1):
    wid = lax.axis_index("s") * NC + lax.axis_index("c")
    base_r = wid * RPW
    is_last = wid == NW - 1
    nrows = jnp.where(is_last, LAST_ROWS, RPW).astype(jnp.int32)
    nvec = nrows // 16
    iota = lax.broadcasted_iota(jnp.int32, (16,), 0)

    ncc = nrows // CCH
    ntail = (nrows - ncc * CCH) // CCT
    hc = ncc // 2

    def cgather(c, buf, sem):
        pltpu.async_copy(mem.at[pl.ds(base_r + c * CCH, CCH)], buf, sem)

    def cscatter(c, buf, sem):
        pltpu.async_copy(buf, new_mem.at[pl.ds(base_r + c * CCH, CCH)], sem)

    def cwait_g(c, buf, sem):
        pltpu.make_async_copy(mem.at[pl.ds(base_r + c * CCH, CCH)], buf, sem).wait()

    def cwait_s(c, buf, sem):
        pltpu.make_async_copy(buf, new_mem.at[pl.ds(base_r + c * CCH, CCH)],
                              sem).wait()

    gbase = wid * BPW

    def gfire(k, buf, sem):
        pltpu.async_copy(mem.at[ids_v.at[pl.ds(gbase + k * GCH, GCH)]], buf, sem)

    def gwait(k, buf, sem):
        pltpu.make_async_copy(mem.at[ids_v.at[pl.ds(gbase + k * GCH, GCH)]],
                              buf, sem).wait()

    def gout(k, buf):
        pltpu.sync_copy(buf, gathered.at[pl.ds(gbase + k * GCH, GCH)])

    cgather(0, cb0, sg0)
    pltpu.async_copy(ts, ts_v, sem_t)
    pltpu.sync_copy(ids, ids_v)
    gfire(0, gbuf, sem_g)
    gfire(1, sbuf, sem_g2)

    def init_body(i, _):
        last_b[pl.ds(i * 16, 16)] = jnp.full((16,), -1, jnp.int32)
        return 0
    lax.fori_loop(0, PT // 16, init_body, 0)

    def scan_body(i, _):
        ids16 = ids_v[pl.ds(i * 16, 16)]
        mine = (ids16 >= base_r) & (ids16 < base_r + nrows)
        plsc.store_scatter(last_b, [ids16 - base_r], i * 16 + iota, mask=mine)
        return 0

    def tbl_body(i, cnt):
        lb = last_b[pl.ds(i * 16, 16)]
        m = lb >= 0
        t = plsc.load_gather(ts_v, [lb], mask=m)
        lu_v[pl.ds(i * 16, 16)] = jnp.where(m, t, jnp.float32(0.0))
        plsc.store_compressed(winb_flat.at[pl.ds(cnt, 16)], lb, mask=m)
        grow = base_r + i * 16 + iota
        plsc.store_compressed(dstr_flat.at[pl.ds(cnt, 16)], grow, mask=m)
        return cnt + jnp.sum(m.astype(jnp.int32))

    def copy_body(c, cnt):
        @pl.when(c % 2 == 0)
        def _():
            @pl.when(c + 1 < ncc)
            def _():
                @pl.when(c >= 1)
                def _():
                    cwait_s(c - 1, cb1, ss1)
                cgather(c + 1, cb1, sg1)

        @pl.when(c % 2 == 1)
        def _():
            @pl.when(c + 1 < ncc)
            def _():
                cwait_s(c - 1, cb0, ss0)
                cgather(c + 1, cb0, sg0)

        sl = jnp.minimum(c, hc) * NSV // hc
        sh = jnp.minimum(c + 1, hc) * NSV // hc
        lax.fori_loop(sl, sh, scan_body, 0)

        @pl.when(c == hc)
        def _():
            pltpu.make_async_copy(ts, ts_v, sem_t).wait()
            gwait(0, gbuf, sem_g)
            gout(0, gbuf)
            gfire(2, gbuf, sem_g)

        @pl.when(c == hc + 3)
        def _():
            gwait(1, sbuf, sem_g2)
            gout(1, sbuf)
            gfire(3, sbuf, sem_g2)

        swl = (jnp.maximum(c, hc) - hc) * nvec // (ncc - hc)
        swh = (jnp.maximum(c + 1, hc) - hc) * nvec // (ncc - hc)
        cnt = lax.fori_loop(swl, swh, tbl_body, cnt)

        @pl.when(c % 2 == 0)
        def _():
            cwait_g(c, cb0, sg0)
            cscatter(c, cb0, ss0)

        @pl.when(c % 2 == 1)
        def _():
            cwait_g(c, cb1, sg1)
            cscatter(c, cb1, ss1)
        return cnt
    cnt = lax.fori_loop(0, ncc, copy_body, jnp.int32(0))

    @pl.when(ncc % 2 == 0)
    def _():
        cwait_s(ncc - 2, cb0, ss0)
        cwait_s(ncc - 1, cb1, ss1)

    @pl.when(ncc % 2 == 1)
    def _():
        cwait_s(ncc - 2, cb1, ss1)
        cwait_s(ncc - 1, cb0, ss0)

    tbase = base_r + ncc * CCH

    def tail_body(t, _):
        pltpu.async_copy(mem.at[pl.ds(tbase + t * CCT, CCT)],
                         cb0.at[pl.ds(0, CCT)], sg0).wait()
        pltpu.async_copy(cb0.at[pl.ds(0, CCT)],
                         new_mem.at[pl.ds(tbase + t * CCT, CCT)], ss0).wait()
        return 0
    lax.fori_loop(0, ntail, tail_body, 0)

    @pl.when(cnt > 0)
    def _():
        lastix = jnp.full((16,), cnt - 1, jnp.int32)
        wpad = plsc.load_gather(winb_flat, [lastix])
        dpad = plsc.load_gather(dstr_flat, [lastix])
        for k in range(SCH // 16):
            winb_flat[pl.ds(cnt + k * 16, 16)] = wpad
            dstr_flat[pl.ds(cnt + k * 16, 16)] = dpad

    nch = (cnt + SCH - 1) // SCH

    def tr_body(j, _):
        v = dstr_flat[pl.ds(j * 16, 16)]
        dstr2d[j // 8, pl.ds((j % 8) * 16, 16)] = v
        return 0
    lax.fori_loop(0, nch * (SCH // 16), tr_body, 0)

    def vg(c, buf, sem):
        pltpu.async_copy(values.at[winb_flat.at[pl.ds(c * SCH, SCH)]], buf, sem)

    def vgw(c, buf, sem):
        pltpu.make_async_copy(values.at[winb_flat.at[pl.ds(c * SCH, SCH)]],
                              buf, sem).wait()

    def rs(c, buf, sem):
        pltpu.async_copy(buf, new_mem.at[dstr2d.at[c]], sem)

    def rsw(c, buf, sem):
        pltpu.make_async_copy(buf, new_mem.at[dstr2d.at[c]], sem).wait()

    @pl.when(nch > 0)
    def _():
        vg(0, cb0, sg0)

    @pl.when(nch > 1)
    def _():
        vg(1, cb1, sg1)

    gwait(2, gbuf, sem_g)
    gout(2, gbuf)
    gwait(3, sbuf, sem_g2)
    gout(3, sbuf)

    @pl.when(jnp.logical_not(is_last))
    def _():
        pltpu.sync_copy(lu_v.at[pl.ds(0, RPW)], new_lu.at[pl.ds(base_r, RPW)])

    @pl.when(is_last)
    def _():
        pltpu.sync_copy(lu_v.at[pl.ds(0, LAST_ROWS)],
                        new_lu.at[pl.ds(base_r, LAST_ROWS)])

    @pl.when(nch > 0)
    def _():
        def sc_body(c, _):
            @pl.when(c % 2 == 0)
            def _():
                vgw(c, cb0, sg0)
                rs(c, cb0, ss0)

                @pl.when(c + 2 < nch)
                def _():
                    rsw(c, cb0, ss0)
                    vg(c + 2, cb0, sg0)

            @pl.when(c % 2 == 1)
            def _():
                vgw(c, cb1, sg1)
                rs(c, cb1, ss1)

                @pl.when(c + 2 < nch)
                def _():
                    rsw(c, cb1, ss1)
                    vg(c + 2, cb1, sg1)
            return 0
        lax.fori_loop(0, nch, sc_body, 0)

        @pl.when(nch == 1)
        def _():
            rsw(0, cb0, ss0)

        @pl.when((nch > 1) & (nch % 2 == 0))
        def _():
            rsw(nch - 2, cb0, ss0)
            rsw(nch - 1, cb1, ss1)

        @pl.when((nch > 1) & (nch % 2 == 1))
        def _():
            rsw(nch - 2, cb1, ss1)
            rsw(nch - 1, cb0, ss0)


def kernel(mem, values, timestamps, node_ids):
    mesh = plsc.VectorSubcoreMesh(core_axis_name="c", subcore_axis_name="s")
    out = pl.kernel(
        _body,
        out_type=(
            jax.ShapeDtypeStruct((B, D), jnp.float32),
            jax.ShapeDtypeStruct((M, D), jnp.float32),
            jax.ShapeDtypeStruct((M,), jnp.float32),
        ),
        mesh=mesh,
        compiler_params=pltpu.CompilerParams(needs_layout_passes=False),
        scratch_types=[
            pltpu.VMEM((B,), jnp.int32),
            pltpu.VMEM((B,), jnp.float32),
            pltpu.VMEM((PT,), jnp.int32),
            pltpu.VMEM((PT,), jnp.float32),
            pltpu.VMEM((LIST_CAP,), jnp.int32),
            pltpu.VMEM((LIST_CAP,), jnp.int32),
            pltpu.VMEM((NCH_MAX, SCH), jnp.int32),
            pltpu.VMEM((GCH, D), jnp.float32),
            pltpu.VMEM((SCH, D), jnp.float32),
            pltpu.VMEM((CCH, D), jnp.float32),
            pltpu.VMEM((CCH, D), jnp.float32),
            pltpu.SemaphoreType.DMA,
            pltpu.SemaphoreType.DMA,
            pltpu.SemaphoreType.DMA,
            pltpu.SemaphoreType.DMA,
            pltpu.SemaphoreType.DMA,
            pltpu.SemaphoreType.DMA,
            pltpu.SemaphoreType.DMA,
        ],
    )(mem, values, timestamps, node_ids)
    return out
